# baseline (speedup 1.0000x reference)
import functools

import jax
import jax.numpy as jnp
import numpy as np
from jax import lax
from jax.experimental import pallas as pl
from jax.experimental.pallas import tpu as pltpu
from jax.experimental.pallas import tpu_sc as plsc

_G = 64
_NPG = 32
_N = _G * _NPG
_EPG = 512
_E = _G * _EPG
_FEA = 128
_C = 256
_L = 4
_T = 10
_H = 8
_DH = _C // _H
_PED = 20
_BH = 64
_MINR = 0.1

_NB = 16
_GPB = _G // _NB
_R = _GPB * _NPG
_EPB = _GPB * _EPG
_ECH = 1024
_W = 128

_dot = functools.partial(lax.dot_general, preferred_element_type=jnp.float32,
                         precision=lax.Precision.HIGHEST)
_dot_fast = functools.partial(lax.dot_general, preferred_element_type=jnp.float32)

_NW = 32
_GPW = _G // _NW
_EPW = _GPW * _EPG
_BINS = _GPW * _NPG * _NPG
_HALF = 16 * _BINS
_IR = 8
_IC = _EPW // _IR


def _hist_body(src_hbm, dst_hbm, out_hbm, src_v, dst_v, idx_v, ones_v,
               zero_v, shared):
    c = lax.axis_index("c")
    s = lax.axis_index("s")
    w = c * 16 + s
    pltpu.sync_copy(src_hbm.at[pl.ds(w * _EPW, _EPW)], src_v)
    pltpu.sync_copy(dst_hbm.at[pl.ds(w * _EPW, _EPW)], dst_v)
    zeros16 = jnp.zeros((16,), jnp.float32)
    ones16 = jnp.ones((16,), jnp.float32)

    def zfill(t, carry):
        zero_v[pl.ds(t * 16, 16)] = zeros16
        return carry

    lax.fori_loop(0, _BINS // 16, zfill, 0)

    def ofill(t, carry):
        ones_v[pl.ds(t * 16, 16)] = ones16
        return carry

    lax.fori_loop(0, _IC // 16, ofill, 0)

    base_row = c * (16 * _GPW * _NPG)
    for r in range(_IR):
        def cbody(k, carry, _r=r):
            sv = src_v[pl.ds(_r * _IC + k * 16, 16)]
            dv = dst_v[pl.ds(_r * _IC + k * 16, 16)]
            idx_v[_r, pl.ds(k * 16, 16)] = ((dv - base_row) * _NPG
                                            + (sv & (_NPG - 1)))
            return carry

        lax.fori_loop(0, _IC // 16, cbody, 0)

    pltpu.sync_copy(zero_v, shared.at[pl.ds(s * _BINS, _BINS)])
    plsc.subcore_barrier()
    for r in range(_IR):
        pltpu.sync_copy(ones_v, shared.at[idx_v.at[r]], add=True)
    plsc.subcore_barrier()
    pltpu.sync_copy(shared.at[pl.ds(s * _BINS, _BINS)],
                    out_hbm.at[pl.ds(c * _HALF + s * _BINS, _BINS)])


def _edge_counts(src, dst):
    mesh = plsc.VectorSubcoreMesh(core_axis_name="c", subcore_axis_name="s")
    run = pl.kernel(
        _hist_body,
        mesh=mesh,
        out_type=jax.ShapeDtypeStruct((2 * _HALF,), jnp.float32),
        scratch_types=[
            pltpu.VMEM((_EPW,), jnp.int32),
            pltpu.VMEM((_EPW,), jnp.int32),
            pltpu.VMEM((_IR, _IC), jnp.int32),
            pltpu.VMEM((_IC,), jnp.float32),
            pltpu.VMEM((_BINS,), jnp.float32),
            pltpu.VMEM_SHARED((_HALF,), jnp.float32),
        ],
    )
    return run(src, dst).reshape(_N, _NPG)


def _fused_body(x_ref, pe_ref, cnt_ref,
                embw_ref, embb_ref, pew_ref, peb_ref,
                bw1_ref, bb1_ref, bw2_ref, bb2_ref,
                g1w_ref, g1b_ref, g2w_ref, g2b_ref, eps_ref,
                sw_ref,
                wq_ref, bq_ref, wk_ref, bk_ref, wv_ref, bv_ref,
                wo_ref, bo_ref,
                linw_ref, linb_ref,
                logits_ref, tr_ref, lg_ref):
    i = pl.program_id(0)
    f32 = jnp.float32

    row_id = lax.broadcasted_iota(jnp.int32, (_R, _R), 0)
    col_id = lax.broadcasted_iota(jnp.int32, (_R, _R), 1)
    same = (row_id // _NPG) == (col_id // _NPG)
    same_f = same.astype(f32)
    eye_f = (row_id == col_id).astype(f32)
    ones_rr = jnp.ones((_R, _R), f32)
    p_sum = ((lax.broadcasted_iota(jnp.int32, (_GPB, _R), 1) // _NPG)
             == lax.broadcasted_iota(jnp.int32, (_GPB, _R), 0)).astype(f32)
    lane_w = lax.broadcasted_iota(jnp.int32, (1, _W), 1)

    def _bcast_cols(col):
        return _dot(ones_rr, eye_f * col, (((1,), (0,)), ((), ())))

    h = _dot(x_ref[...], embw_ref[...], (((1,), (0,)), ((), ()))) + embb_ref[...]
    h = h + _dot(pe_ref[...], pew_ref[...], (((1,), (0,)), ((), ()))) + peb_ref[...]

    pooled = _dot(same_f, h, (((1,), (0,)), ((), ()))) * (1.0 / _NPG)
    hid = jnp.maximum(
        _dot(pooled, bw1_ref[...], (((1,), (0,)), ((), ()))) + bb1_ref[...], 0.0)
    bd = _dot(hid, bw2_ref[...], (((1,), (0,)), ((), ()))) + bb2_ref[...]
    trs = _MINR + (1.0 - _MINR) * jax.nn.sigmoid(bd)
    lgs = jax.nn.sigmoid(bd)
    tr_ref[...] = (_dot(p_sum, trs, (((1,), (0,)), ((), ()))) * (1.0 / _NPG))[None]
    lg_ref[...] = (_dot(p_sum, lgs, (((1,), (0,)), ((), ()))) * (1.0 / _NPG))[None]

    tmat = ((lax.broadcasted_iota(jnp.int32, (_NPG, _R), 1) % _NPG)
            == lax.broadcasted_iota(jnp.int32, (_NPG, _R), 0)).astype(f32)
    adj = _dot(cnt_ref[...], tmat, (((1,), (0,)), ((), ()))) * same_f

    neg = jnp.float32(-1e30)
    inv_sqrt = jnp.float32(1.0 / np.sqrt(_DH))

    for l in range(_L):
        agg = _dot(adj, h, (((1,), (0,)), ((), ())))
        eps = eps_ref[0:1, l:l + 1]
        z = h + eps * h + agg
        t1 = jnp.maximum(
            _dot(z, g1w_ref[l], (((1,), (0,)), ((), ()))) + g1b_ref[l], 0.0)
        h_local = h + _dot(t1, g2w_ref[l], (((1,), (0,)), ((), ()))) + g2b_ref[l]

        s_col = jnp.sum(h * sw_ref[l], axis=1, keepdims=True)
        smat = _bcast_cols(s_col)
        lt = smat < s_col
        eq = smat == s_col
        rank_col = jnp.sum(
            (same & (lt | (eq & (col_id < row_id)))).astype(f32),
            axis=1, keepdims=True)

        trl = jnp.sum(trs * (lane_w == l).astype(f32), axis=1, keepdims=True)
        lgl = jnp.sum(lgs * (lane_w == (_L + l)).astype(f32), axis=1, keepdims=True)
        idx_col = jnp.clip(jnp.floor((1.0 - trl) * float(_NPG - 1)),
                           0.0, float(_NPG - 1))
        dmat = _bcast_cols(rank_col - idx_col)
        selmat = (dmat == 0.0).astype(f32)
        thr_col = jnp.sum(same_f * smat * selmat, axis=1, keepdims=True)
        m_col = jax.nn.sigmoid(s_col - thr_col)

        bias = jnp.where(same, _bcast_cols(jnp.log(m_col + 1e-6)), neg)
        q = _dot(h, wq_ref[l], (((1,), (0,)), ((), ()))) + bq_ref[l]
        k = _dot(h, wk_ref[l], (((1,), (0,)), ((), ()))) + bk_ref[l]
        v = _dot(h, wv_ref[l], (((1,), (0,)), ((), ()))) + bv_ref[l]
        ovs = []
        for hh in range(_H):
            qh = q[:, hh * _DH:(hh + 1) * _DH]
            kh = k[:, hh * _DH:(hh + 1) * _DH]
            vh = v[:, hh * _DH:(hh + 1) * _DH]
            sc = _dot(qh, kh, (((1,), (1,)), ((), ()))) * inv_sqrt + bias
            p = jax.nn.softmax(sc, axis=-1)
            ovs.append(_dot(p, vh, (((1,), (0,)), ((), ()))))
        o_mat = jnp.concatenate(ovs, axis=1)
        o = (_dot(o_mat, wo_ref[l], (((1,), (0,)), ((), ()))) + bo_ref[l]) * m_col
        hsum = h_local + h + lgl * o
        mu = jnp.mean(hsum, axis=-1, keepdims=True)
        d = hsum - mu
        var = jnp.mean(d * d, axis=-1, keepdims=True)
        h = d / jnp.sqrt(var + 1e-5)

    out = _dot(h, linw_ref[...], (((1,), (0,)), ((), ())))
    logits_ref[...] = (_dot(p_sum, out, (((1,), (0,)), ((), ())))
                       + linb_ref[...])[None]


def _full(shape):
    nd = len(shape)
    return pl.BlockSpec(shape, lambda i, _nd=nd: (0,) * _nd)


def _pad_lanes(a, w=_W):
    return jnp.pad(a, ((0, 0),) * (a.ndim - 1) + ((0, w - a.shape[-1]),))


def kernel(x, pe, edge_index, edge_attr, batch, params):
    del edge_attr, batch
    f32 = jnp.float32
    lyr = params['layers']

    embw, embb = params['node_emb']
    pew, peb = params['pe_lin']
    bw1, bb1 = params['budget_w1']
    bw2, bb2 = params['budget_w2']
    linw, linb = params['lin']

    pe_p = _pad_lanes(pe)
    pew_p = jnp.pad(pew, ((0, 128 - _PED), (0, 0)))
    counts = _edge_counts(edge_index[0], edge_index[1])

    bw1_p = _pad_lanes(bw1)
    bb1_p = _pad_lanes(bb1[None, :])
    bw2_p = jnp.pad(bw2, ((0, _W - _BH), (0, _W - 2 * _L)))
    bb2_p = _pad_lanes(bb2[None, :])
    linw_p = _pad_lanes(linw)
    linb_p = _pad_lanes(linb[None, :])

    g1w = jnp.stack([lp['gin_w1'][0] for lp in lyr])
    g1b = jnp.stack([lp['gin_w1'][1] for lp in lyr])[:, None, :]
    g2w = jnp.stack([lp['gin_w2'][0] for lp in lyr])
    g2b = jnp.stack([lp['gin_w2'][1] for lp in lyr])[:, None, :]
    eps = jnp.stack([lp['eps'] for lp in lyr]).reshape(1, _L)
    sw = jnp.stack([lp['score'][0][:, 0] for lp in lyr])[:, None, :]

    wq = jnp.stack([lp['wq'][0] for lp in lyr])
    bq = jnp.stack([lp['wq'][1] for lp in lyr])[:, None, :]
    wk = jnp.stack([lp['wk'][0] for lp in lyr])
    bk = jnp.stack([lp['wk'][1] for lp in lyr])[:, None, :]
    wv = jnp.stack([lp['wv'][0] for lp in lyr])
    bv = jnp.stack([lp['wv'][1] for lp in lyr])[:, None, :]
    wo = jnp.stack([lp['wo'][0] for lp in lyr])
    bo = jnp.stack([lp['wo'][1] for lp in lyr])[:, None, :]

    operands = (
        x, pe_p, counts,
        embw, embb[None, :], pew_p, peb[None, :],
        bw1_p, bb1_p, bw2_p, bb2_p,
        g1w, g1b, g2w, g2b, eps,
        sw,
        wq, bq, wk, bk, wv, bv,
        wo, bo,
        linw_p, linb_p,
    )

    in_specs = [
        pl.BlockSpec((_R, _FEA), lambda i: (i, 0)),
        pl.BlockSpec((_R, 128), lambda i: (i, 0)),
        pl.BlockSpec((_R, _NPG), lambda i: (i, 0)),
    ] + [_full(op.shape) for op in operands[3:]]

    out_shape = (
        jax.ShapeDtypeStruct((_NB, _GPB, _W), f32),
        jax.ShapeDtypeStruct((_NB, _GPB, _W), f32),
        jax.ShapeDtypeStruct((_NB, _GPB, _W), f32),
    )
    out_specs = (
        pl.BlockSpec((1, _GPB, _W), lambda i: (i, 0, 0)),
        pl.BlockSpec((1, _GPB, _W), lambda i: (i, 0, 0)),
        pl.BlockSpec((1, _GPB, _W), lambda i: (i, 0, 0)),
    )

    logits_p, tr_p, lg_p = pl.pallas_call(
        _fused_body,
        grid=(_NB,),
        in_specs=in_specs,
        out_specs=out_specs,
        out_shape=out_shape,
    )(*operands)

    logits = logits_p.reshape(_G, _W)[:, :_T]
    tr = tr_p.reshape(_G, _W)[:, :_L]
    lg = lg_p.reshape(_G, _W)[:, _L:2 * _L]

    costs = [tr[:, l].mean() ** 2 * lg[:, l].mean() for l in range(_L)]
    dense_macs = float(_G * _H * _NPG * _NPG * _DH * 2 + 6 * _N * _C * _C)
    avg_compute = sum(costs) / float(_L)
    total_actual = sum(costs) * dense_macs
    total_dense = jnp.float32(dense_macs * _L)
    return (logits, avg_compute, tr, lg, total_actual, total_dense)


# speedup vs baseline: 1.2145x; 1.2145x over previous
import functools

import jax
import jax.numpy as jnp
import numpy as np
from jax import lax
from jax.experimental import pallas as pl
from jax.experimental.pallas import tpu as pltpu
from jax.experimental.pallas import tpu_sc as plsc

_G = 64
_NPG = 32
_N = _G * _NPG
_EPG = 512
_E = _G * _EPG
_FEA = 128
_C = 256
_L = 4
_T = 10
_H = 8
_DH = _C // _H
_PED = 20
_BH = 64
_MINR = 0.1

_NB = 16
_GPB = _G // _NB
_R = _GPB * _NPG
_W = 128

_dot = functools.partial(lax.dot_general, preferred_element_type=jnp.float32,
                         precision=lax.Precision.HIGHEST)
_dot_fast = functools.partial(lax.dot_general, preferred_element_type=jnp.float32)

_NW = 32
_GPW = _G // _NW
_EPW = _GPW * _EPG
_BINS = _GPW * _NPG * _NPG
_HALF = 16 * _BINS
_IR = 8
_IC = _EPW // _IR


def _hist_body(src_hbm, dst_hbm, out_hbm, src_v, dst_v, idx_v, ones_v,
               zero_v, shared):
    c = lax.axis_index("c")
    s = lax.axis_index("s")
    w = c * 16 + s
    pltpu.sync_copy(src_hbm.at[pl.ds(w * _EPW, _EPW)], src_v)
    pltpu.sync_copy(dst_hbm.at[pl.ds(w * _EPW, _EPW)], dst_v)
    zeros16 = jnp.zeros((16,), jnp.float32)
    ones16 = jnp.ones((16,), jnp.float32)

    def zfill(t, carry):
        zero_v[pl.ds(t * 16, 16)] = zeros16
        return carry

    lax.fori_loop(0, _BINS // 16, zfill, 0)

    def ofill(t, carry):
        ones_v[pl.ds(t * 16, 16)] = ones16
        return carry

    lax.fori_loop(0, _IC // 16, ofill, 0)

    base_row = c * (16 * _GPW * _NPG)
    for r in range(_IR):
        def cbody(k, carry, _r=r):
            sv = src_v[pl.ds(_r * _IC + k * 16, 16)]
            dv = dst_v[pl.ds(_r * _IC + k * 16, 16)]
            idx_v[_r, pl.ds(k * 16, 16)] = ((dv - base_row) * _NPG
                                            + (sv & (_NPG - 1)))
            return carry

        lax.fori_loop(0, _IC // 16, cbody, 0)

    pltpu.sync_copy(zero_v, shared.at[pl.ds(s * _BINS, _BINS)])
    plsc.subcore_barrier()
    for r in range(_IR):
        pltpu.sync_copy(ones_v, shared.at[idx_v.at[r]], add=True)
    plsc.subcore_barrier()
    pltpu.sync_copy(shared.at[pl.ds(s * _BINS, _BINS)],
                    out_hbm.at[pl.ds(c * _HALF + s * _BINS, _BINS)])


def _edge_counts(src, dst):
    mesh = plsc.VectorSubcoreMesh(core_axis_name="c", subcore_axis_name="s")
    run = pl.kernel(
        _hist_body,
        mesh=mesh,
        out_type=jax.ShapeDtypeStruct((2 * _HALF,), jnp.float32),
        scratch_types=[
            pltpu.VMEM((_EPW,), jnp.int32),
            pltpu.VMEM((_EPW,), jnp.int32),
            pltpu.VMEM((_IR, _IC), jnp.int32),
            pltpu.VMEM((_IC,), jnp.float32),
            pltpu.VMEM((_BINS,), jnp.float32),
            pltpu.VMEM_SHARED((_HALF,), jnp.float32),
        ],
    )
    return run(src, dst).reshape(_N, _NPG)


_BF = jnp.bfloat16
_CT = (((1,), (0,)), ((), ()))
_CTT = (((1,), (1,)), ((), ()))


def _sp(a):
    hi = a.astype(_BF)
    lo = (a - hi.astype(jnp.float32)).astype(_BF)
    return hi, lo


def _d3(ahi, alo, whi, wlo, dims=_CT):
    return (_dot_fast(ahi, whi, dims) + _dot_fast(ahi, wlo, dims)
            + _dot_fast(alo, whi, dims))


def _fused_body(xh_ref, xl_ref, peh_ref, pel_ref, cnt_ref,
                embw_ref, embb_ref, pew_ref, peb_ref,
                bw1_ref, bb1_ref, bw2_ref, bb2_ref,
                g1w_ref, g1b_ref, g2w_ref, g2b_ref, eps_ref,
                sw_ref,
                wq_ref, bq_ref, wk_ref, bk_ref, wv_ref, bv_ref,
                wo_ref, bo_ref,
                linw_ref, linb_ref,
                logits_ref, tr_ref, lg_ref):
    f32 = jnp.float32

    row_id = lax.broadcasted_iota(jnp.int32, (_R, _R), 0)
    col_id = lax.broadcasted_iota(jnp.int32, (_R, _R), 1)
    same = (row_id // _NPG) == (col_id // _NPG)
    same_f = same.astype(f32)
    same_b = same.astype(_BF)
    eye_f = (row_id == col_id).astype(f32)
    ones_rr = jnp.ones((_R, _R), f32)
    p_sum = ((lax.broadcasted_iota(jnp.int32, (_GPB, _R), 1) // _NPG)
             == lax.broadcasted_iota(jnp.int32, (_GPB, _R), 0)).astype(f32)
    lane_w = lax.broadcasted_iota(jnp.int32, (1, _W), 1)

    def _bcast_cols(col):
        return _dot(ones_rr, eye_f * col, _CT)

    h = (_d3(xh_ref[...], xl_ref[...], embw_ref[0], embw_ref[1]) + embb_ref[...]
         + _d3(peh_ref[...], pel_ref[...], pew_ref[0], pew_ref[1])
         + peb_ref[...])

    h_hi, h_lo = _sp(h)
    pooled = (_dot_fast(same_b, h_hi, _CT)
              + _dot_fast(same_b, h_lo, _CT)) * (1.0 / _NPG)
    hid = jnp.maximum(
        _d3(*_sp(pooled), bw1_ref[0], bw1_ref[1]) + bb1_ref[...], 0.0)
    bd = _d3(*_sp(hid), bw2_ref[0], bw2_ref[1]) + bb2_ref[...]
    trs = _MINR + (1.0 - _MINR) * jax.nn.sigmoid(bd)
    lgs = jax.nn.sigmoid(bd)
    tr_ref[...] = (_dot(p_sum, trs, _CT) * (1.0 / _NPG))[None]
    lg_ref[...] = (_dot(p_sum, lgs, _CT) * (1.0 / _NPG))[None]

    tmat = ((lax.broadcasted_iota(jnp.int32, (_NPG, _R), 1) % _NPG)
            == lax.broadcasted_iota(jnp.int32, (_NPG, _R), 0)).astype(_BF)
    cnt_hi, cnt_lo = _sp(cnt_ref[...])
    adj = (_dot_fast(cnt_hi, tmat, _CT)
           + _dot_fast(cnt_lo, tmat, _CT)) * same_f
    adj_hi, adj_lo = _sp(adj)

    neg = jnp.float32(-1e30)
    inv_sqrt = jnp.float32(1.0 / np.sqrt(_DH))

    for l in range(_L):
        agg = _d3(adj_hi, adj_lo, h_hi, h_lo)
        eps = eps_ref[0:1, l:l + 1]
        z = h + eps * h + agg
        t1 = jnp.maximum(
            _d3(*_sp(z), g1w_ref[0, l], g1w_ref[1, l]) + g1b_ref[l], 0.0)
        h_local = (h + _d3(*_sp(t1), g2w_ref[0, l], g2w_ref[1, l])
                   + g2b_ref[l])

        s_col = jnp.sum(h * sw_ref[l], axis=1, keepdims=True)
        smat = _bcast_cols(s_col)
        lt = smat < s_col
        eq = smat == s_col
        rank_col = jnp.sum(
            (same & (lt | (eq & (col_id < row_id)))).astype(f32),
            axis=1, keepdims=True)

        trl = jnp.sum(trs * (lane_w == l).astype(f32), axis=1, keepdims=True)
        lgl = jnp.sum(lgs * (lane_w == (_L + l)).astype(f32), axis=1, keepdims=True)
        idx_col = jnp.clip(jnp.floor((1.0 - trl) * float(_NPG - 1)),
                           0.0, float(_NPG - 1))
        dmat = _bcast_cols(rank_col - idx_col)
        selmat = (dmat == 0.0).astype(f32)
        thr_col = jnp.sum(same_f * smat * selmat, axis=1, keepdims=True)
        m_col = jax.nn.sigmoid(s_col - thr_col)

        bias = jnp.where(same, _bcast_cols(jnp.log(m_col + 1e-6)), neg)
        q = _d3(h_hi, h_lo, wq_ref[0, l], wq_ref[1, l]) + bq_ref[l]
        k = _d3(h_hi, h_lo, wk_ref[0, l], wk_ref[1, l]) + bk_ref[l]
        v = _d3(h_hi, h_lo, wv_ref[0, l], wv_ref[1, l]) + bv_ref[l]
        q_hi, q_lo = _sp(q)
        k_hi, k_lo = _sp(k)
        v_hi, v_lo = _sp(v)
        ovs = []
        for hh in range(_H):
            sl = slice(hh * _DH, (hh + 1) * _DH)
            sc = (_dot_fast(q_hi[:, sl], k_hi[:, sl], _CTT)
                  + _dot_fast(q_hi[:, sl], k_lo[:, sl], _CTT)
                  + _dot_fast(q_lo[:, sl], k_hi[:, sl], _CTT)) * inv_sqrt + bias
            p = jax.nn.softmax(sc, axis=-1)
            p_hi, p_lo = _sp(p)
            ovs.append(_dot_fast(p_hi, v_hi[:, sl], _CT)
                       + _dot_fast(p_hi, v_lo[:, sl], _CT)
                       + _dot_fast(p_lo, v_hi[:, sl], _CT))
        o_mat = jnp.concatenate(ovs, axis=1)
        o = (_d3(*_sp(o_mat), wo_ref[0, l], wo_ref[1, l])
             + bo_ref[l]) * m_col
        hsum = h_local + h + lgl * o
        mu = jnp.mean(hsum, axis=-1, keepdims=True)
        d = hsum - mu
        var = jnp.mean(d * d, axis=-1, keepdims=True)
        h = d / jnp.sqrt(var + 1e-5)
        h_hi, h_lo = _sp(h)

    out = _d3(h_hi, h_lo, linw_ref[0], linw_ref[1])
    logits_ref[...] = (_dot(p_sum, out, _CT) + linb_ref[...])[None]


def _full(shape):
    nd = len(shape)
    return pl.BlockSpec(shape, lambda i, _nd=nd: (0,) * _nd)


def _pad_lanes(a, w=_W):
    return jnp.pad(a, ((0, 0),) * (a.ndim - 1) + ((0, w - a.shape[-1]),))


def _split2(a):
    hi = a.astype(_BF)
    lo = (a - hi.astype(jnp.float32)).astype(_BF)
    return jnp.stack([hi, lo])


def kernel(x, pe, edge_index, edge_attr, batch, params):
    del edge_attr, batch
    f32 = jnp.float32
    lyr = params['layers']

    embw, embb = params['node_emb']
    pew, peb = params['pe_lin']
    bw1, bb1 = params['budget_w1']
    bw2, bb2 = params['budget_w2']
    linw, linb = params['lin']

    x_hi = x.astype(_BF)
    x_lo = (x - x_hi.astype(f32)).astype(_BF)
    pe_p = _pad_lanes(pe)
    pe_hi = pe_p.astype(_BF)
    pe_lo = (pe_p - pe_hi.astype(f32)).astype(_BF)
    pew_p = jnp.pad(pew, ((0, 128 - _PED), (0, 0)))
    counts = _edge_counts(edge_index[0], edge_index[1])

    bw1_p = _pad_lanes(bw1)
    bb1_p = _pad_lanes(bb1[None, :])
    bw2_p = jnp.pad(bw2, ((0, _W - _BH), (0, _W - 2 * _L)))
    bb2_p = _pad_lanes(bb2[None, :])
    linw_p = _pad_lanes(linw)
    linb_p = _pad_lanes(linb[None, :])

    g1w = jnp.stack([lp['gin_w1'][0] for lp in lyr])
    g1b = jnp.stack([lp['gin_w1'][1] for lp in lyr])[:, None, :]
    g2w = jnp.stack([lp['gin_w2'][0] for lp in lyr])
    g2b = jnp.stack([lp['gin_w2'][1] for lp in lyr])[:, None, :]
    eps = jnp.stack([lp['eps'] for lp in lyr]).reshape(1, _L)
    sw = jnp.stack([lp['score'][0][:, 0] for lp in lyr])[:, None, :]

    wq = jnp.stack([lp['wq'][0] for lp in lyr])
    bq = jnp.stack([lp['wq'][1] for lp in lyr])[:, None, :]
    wk = jnp.stack([lp['wk'][0] for lp in lyr])
    bk = jnp.stack([lp['wk'][1] for lp in lyr])[:, None, :]
    wv = jnp.stack([lp['wv'][0] for lp in lyr])
    bv = jnp.stack([lp['wv'][1] for lp in lyr])[:, None, :]
    wo = jnp.stack([lp['wo'][0] for lp in lyr])
    bo = jnp.stack([lp['wo'][1] for lp in lyr])[:, None, :]

    operands = (
        x_hi, x_lo, pe_hi, pe_lo, counts,
        _split2(embw), embb[None, :], _split2(pew_p), peb[None, :],
        _split2(bw1_p), bb1_p, _split2(bw2_p), bb2_p,
        _split2(g1w), g1b, _split2(g2w), g2b, eps,
        sw,
        _split2(wq), bq, _split2(wk), bk, _split2(wv), bv,
        _split2(wo), bo,
        _split2(linw_p), linb_p,
    )

    in_specs = [
        pl.BlockSpec((_R, _FEA), lambda i: (i, 0)),
        pl.BlockSpec((_R, _FEA), lambda i: (i, 0)),
        pl.BlockSpec((_R, 128), lambda i: (i, 0)),
        pl.BlockSpec((_R, 128), lambda i: (i, 0)),
        pl.BlockSpec((_R, _NPG), lambda i: (i, 0)),
    ] + [_full(op.shape) for op in operands[5:]]

    out_shape = (
        jax.ShapeDtypeStruct((_NB, _GPB, _W), f32),
        jax.ShapeDtypeStruct((_NB, _GPB, _W), f32),
        jax.ShapeDtypeStruct((_NB, _GPB, _W), f32),
    )
    out_specs = (
        pl.BlockSpec((1, _GPB, _W), lambda i: (i, 0, 0)),
        pl.BlockSpec((1, _GPB, _W), lambda i: (i, 0, 0)),
        pl.BlockSpec((1, _GPB, _W), lambda i: (i, 0, 0)),
    )

    logits_p, tr_p, lg_p = pl.pallas_call(
        _fused_body,
        grid=(_NB,),
        in_specs=in_specs,
        out_specs=out_specs,
        out_shape=out_shape,
    )(*operands)

    logits = logits_p.reshape(_G, _W)[:, :_T]
    tr = tr_p.reshape(_G, _W)[:, :_L]
    lg = lg_p.reshape(_G, _W)[:, _L:2 * _L]

    costs = [tr[:, l].mean() ** 2 * lg[:, l].mean() for l in range(_L)]
    dense_macs = float(_G * _H * _NPG * _NPG * _DH * 2 + 6 * _N * _C * _C)
    avg_compute = sum(costs) / float(_L)
    total_actual = sum(costs) * dense_macs
    total_dense = jnp.float32(dense_macs * _L)
    return (logits, avg_compute, tr, lg, total_actual, total_dense)


# speedup vs baseline: 1.3425x; 1.1054x over previous
import functools

import jax
import jax.numpy as jnp
import numpy as np
from jax import lax
from jax.experimental import pallas as pl
from jax.experimental.pallas import tpu as pltpu
from jax.experimental.pallas import tpu_sc as plsc

_G = 64
_NPG = 32
_N = _G * _NPG
_EPG = 512
_E = _G * _EPG
_FEA = 128
_C = 256
_L = 4
_T = 10
_H = 8
_DH = _C // _H
_PED = 20
_BH = 64
_MINR = 0.1

_NB = 16
_GPB = _G // _NB
_R = _GPB * _NPG
_W = 128

_dot = functools.partial(lax.dot_general, preferred_element_type=jnp.float32,
                         precision=lax.Precision.HIGHEST)
_dot_fast = functools.partial(lax.dot_general, preferred_element_type=jnp.float32)

_NW = 32
_GPW = _G // _NW
_EPW = _GPW * _EPG
_BINS = _GPW * _NPG * _NPG
_HALF = 16 * _BINS
_IR = 8
_IC = _EPW // _IR


def _hist_body(src_hbm, dst_hbm, out_hbm, src_v, dst_v, idx_v, ones_v,
               zero_v, shared):
    c = lax.axis_index("c")
    s = lax.axis_index("s")
    w = c * 16 + s
    pltpu.sync_copy(src_hbm.at[pl.ds(w * _EPW, _EPW)], src_v)
    pltpu.sync_copy(dst_hbm.at[pl.ds(w * _EPW, _EPW)], dst_v)
    zeros16 = jnp.zeros((16,), jnp.float32)
    ones16 = jnp.ones((16,), jnp.float32)

    def zfill(t, carry):
        zero_v[pl.ds(t * 16, 16)] = zeros16
        return carry

    lax.fori_loop(0, _BINS // 16, zfill, 0)

    def ofill(t, carry):
        ones_v[pl.ds(t * 16, 16)] = ones16
        return carry

    lax.fori_loop(0, _IC // 16, ofill, 0)

    base_row = c * (16 * _GPW * _NPG)
    for r in range(_IR):
        def cbody(k, carry, _r=r):
            sv = src_v[pl.ds(_r * _IC + k * 16, 16)]
            dv = dst_v[pl.ds(_r * _IC + k * 16, 16)]
            idx_v[_r, pl.ds(k * 16, 16)] = ((dv - base_row) * _NPG
                                            + (sv & (_NPG - 1)))
            return carry

        lax.fori_loop(0, _IC // 16, cbody, 0)

    pltpu.sync_copy(zero_v, shared.at[pl.ds(s * _BINS, _BINS)])
    plsc.subcore_barrier()
    for r in range(_IR):
        pltpu.sync_copy(ones_v, shared.at[idx_v.at[r]], add=True)
    plsc.subcore_barrier()
    pltpu.sync_copy(shared.at[pl.ds(s * _BINS, _BINS)],
                    out_hbm.at[pl.ds(c * _HALF + s * _BINS, _BINS)])


def _edge_counts(src, dst):
    mesh = plsc.VectorSubcoreMesh(core_axis_name="c", subcore_axis_name="s")
    run = pl.kernel(
        _hist_body,
        mesh=mesh,
        out_type=jax.ShapeDtypeStruct((2 * _HALF,), jnp.float32),
        scratch_types=[
            pltpu.VMEM((_EPW,), jnp.int32),
            pltpu.VMEM((_EPW,), jnp.int32),
            pltpu.VMEM((_IR, _IC), jnp.int32),
            pltpu.VMEM((_IC,), jnp.float32),
            pltpu.VMEM((_BINS,), jnp.float32),
            pltpu.VMEM_SHARED((_HALF,), jnp.float32),
        ],
    )
    return run(src, dst).reshape(_N, _NPG)


_BF = jnp.bfloat16
_CT = (((1,), (0,)), ((), ()))
_CTT = (((1,), (1,)), ((), ()))


def _sp(a):
    hi = a.astype(_BF)
    lo = (a - hi.astype(jnp.float32)).astype(_BF)
    return hi, lo


def _d3(ahi, alo, whi, wlo, dims=_CT):
    return (_dot_fast(ahi, whi, dims) + _dot_fast(ahi, wlo, dims)
            + _dot_fast(alo, whi, dims))


def _fused_body(xh_ref, xl_ref, peh_ref, pel_ref, cnt_ref,
                embw_ref, embb_ref, pew_ref, peb_ref,
                bw1_ref, bb1_ref, bw2_ref, bb2_ref,
                g1w_ref, g1b_ref, g2w_ref, g2b_ref, eps_ref,
                sw_ref,
                wq_ref, bq_ref, wk_ref, bk_ref, wv_ref, bv_ref,
                wo_ref, bo_ref,
                linw_ref, linb_ref,
                logits_ref, tr_ref, lg_ref):
    f32 = jnp.float32

    row_id = lax.broadcasted_iota(jnp.int32, (_R, _R), 0)
    col_id = lax.broadcasted_iota(jnp.int32, (_R, _R), 1)
    same = (row_id // _NPG) == (col_id // _NPG)
    same_f = same.astype(f32)
    same_b = same.astype(_BF)
    ones_rr = jnp.ones((_R, _R), f32)
    p_sum = ((lax.broadcasted_iota(jnp.int32, (_GPB, _R), 1) // _NPG)
             == lax.broadcasted_iota(jnp.int32, (_GPB, _R), 0)).astype(f32)
    lane_w = lax.broadcasted_iota(jnp.int32, (1, _W), 1)

    ones_b = ones_rr.astype(_BF)
    eye_f = (row_id == col_id).astype(f32)

    def _bc_exact(col):
        d = eye_f * col
        hi = d.astype(_BF)
        r1 = d - hi.astype(jnp.float32)
        mid = r1.astype(_BF)
        lo = (r1 - mid.astype(jnp.float32)).astype(_BF)
        return ((_dot_fast(ones_b, hi, _CT) + _dot_fast(ones_b, mid, _CT))
                + _dot_fast(ones_b, lo, _CT))

    def _bc_int(col):
        return _dot_fast(ones_b, (eye_f * col).astype(_BF), _CT)

    def _bc_approx(col):
        d = eye_f * col
        hi = d.astype(_BF)
        lo = (d - hi.astype(jnp.float32)).astype(_BF)
        return _dot_fast(ones_b, hi, _CT) + _dot_fast(ones_b, lo, _CT)

    h = (_d3(xh_ref[...], xl_ref[...], embw_ref[0], embw_ref[1]) + embb_ref[...]
         + _d3(peh_ref[...], pel_ref[...], pew_ref[0], pew_ref[1])
         + peb_ref[...])

    h_hi, h_lo = _sp(h)
    pooled = (_dot_fast(same_b, h_hi, _CT)
              + _dot_fast(same_b, h_lo, _CT)) * (1.0 / _NPG)
    hid = jnp.maximum(
        _d3(*_sp(pooled), bw1_ref[0], bw1_ref[1]) + bb1_ref[...], 0.0)
    bd = _d3(*_sp(hid), bw2_ref[0], bw2_ref[1]) + bb2_ref[...]
    trs = _MINR + (1.0 - _MINR) * jax.nn.sigmoid(bd)
    lgs = jax.nn.sigmoid(bd)
    tr_ref[...] = (_dot(p_sum, trs, _CT) * (1.0 / _NPG))[None]
    lg_ref[...] = (_dot(p_sum, lgs, _CT) * (1.0 / _NPG))[None]

    tmat = ((lax.broadcasted_iota(jnp.int32, (_NPG, _R), 1) % _NPG)
            == lax.broadcasted_iota(jnp.int32, (_NPG, _R), 0)).astype(_BF)
    cnt_hi, cnt_lo = _sp(cnt_ref[...])
    adj = (_dot_fast(cnt_hi, tmat, _CT)
           + _dot_fast(cnt_lo, tmat, _CT)) * same_f
    adj_hi, adj_lo = _sp(adj)

    neg = jnp.float32(-1e30)
    inv_sqrt = jnp.float32(1.0 / np.sqrt(_DH))

    for l in range(_L):
        agg = _d3(adj_hi, adj_lo, h_hi, h_lo)
        eps = eps_ref[0:1, l:l + 1]
        z = h + eps * h + agg
        t1 = jnp.maximum(
            _d3(*_sp(z), g1w_ref[0, l], g1w_ref[1, l]) + g1b_ref[l], 0.0)
        h_local = (h + _d3(*_sp(t1), g2w_ref[0, l], g2w_ref[1, l])
                   + g2b_ref[l])

        s_col = jnp.sum(h * sw_ref[l], axis=1, keepdims=True)
        smat = _bc_exact(s_col)
        lt = smat < s_col
        eq = smat == s_col
        rank_col = jnp.sum(
            (same & (lt | (eq & (col_id < row_id)))).astype(f32),
            axis=1, keepdims=True)

        trl = jnp.sum(trs * (lane_w == l).astype(f32), axis=1, keepdims=True)
        lgl = jnp.sum(lgs * (lane_w == (_L + l)).astype(f32), axis=1, keepdims=True)
        idx_col = jnp.clip(jnp.floor((1.0 - trl) * float(_NPG - 1)),
                           0.0, float(_NPG - 1))
        dmat = _bc_int(rank_col - idx_col)
        selmat = (dmat == 0.0).astype(f32)
        thr_col = jnp.sum(same_f * smat * selmat, axis=1, keepdims=True)
        m_col = jax.nn.sigmoid(s_col - thr_col)

        bias = jnp.where(same, _bc_approx(jnp.log(m_col + 1e-6)), neg)
        q = _d3(h_hi, h_lo, wq_ref[0, l], wq_ref[1, l]) + bq_ref[l]
        k = _d3(h_hi, h_lo, wk_ref[0, l], wk_ref[1, l]) + bk_ref[l]
        v = _d3(h_hi, h_lo, wv_ref[0, l], wv_ref[1, l]) + bv_ref[l]
        q_hi, q_lo = _sp(q)
        k_hi, k_lo = _sp(k)
        v_hi, v_lo = _sp(v)
        ovs = []
        for hh in range(_H):
            sl = slice(hh * _DH, (hh + 1) * _DH)
            sc = (_dot_fast(q_hi[:, sl], k_hi[:, sl], _CTT)
                  + _dot_fast(q_hi[:, sl], k_lo[:, sl], _CTT)
                  + _dot_fast(q_lo[:, sl], k_hi[:, sl], _CTT)) * inv_sqrt + bias
            p = jax.nn.softmax(sc, axis=-1)
            p_hi, p_lo = _sp(p)
            ovs.append(_dot_fast(p_hi, v_hi[:, sl], _CT)
                       + _dot_fast(p_hi, v_lo[:, sl], _CT)
                       + _dot_fast(p_lo, v_hi[:, sl], _CT))
        o_mat = jnp.concatenate(ovs, axis=1)
        o = (_d3(*_sp(o_mat), wo_ref[0, l], wo_ref[1, l])
             + bo_ref[l]) * m_col
        hsum = h_local + h + lgl * o
        mu = jnp.mean(hsum, axis=-1, keepdims=True)
        d = hsum - mu
        var = jnp.mean(d * d, axis=-1, keepdims=True)
        h = d / jnp.sqrt(var + 1e-5)
        h_hi, h_lo = _sp(h)

    out = _d3(h_hi, h_lo, linw_ref[0], linw_ref[1])
    logits_ref[...] = (_dot(p_sum, out, _CT) + linb_ref[...])[None]


def _full(shape):
    nd = len(shape)
    return pl.BlockSpec(shape, lambda i, _nd=nd: (0,) * _nd)


def _pad_lanes(a, w=_W):
    return jnp.pad(a, ((0, 0),) * (a.ndim - 1) + ((0, w - a.shape[-1]),))


def _split2(a):
    hi = a.astype(_BF)
    lo = (a - hi.astype(jnp.float32)).astype(_BF)
    return jnp.stack([hi, lo])


def kernel(x, pe, edge_index, edge_attr, batch, params):
    del edge_attr, batch
    f32 = jnp.float32
    lyr = params['layers']

    embw, embb = params['node_emb']
    pew, peb = params['pe_lin']
    bw1, bb1 = params['budget_w1']
    bw2, bb2 = params['budget_w2']
    linw, linb = params['lin']

    x_hi = x.astype(_BF)
    x_lo = (x - x_hi.astype(f32)).astype(_BF)
    pe_p = _pad_lanes(pe)
    pe_hi = pe_p.astype(_BF)
    pe_lo = (pe_p - pe_hi.astype(f32)).astype(_BF)
    pew_p = jnp.pad(pew, ((0, 128 - _PED), (0, 0)))
    counts = _edge_counts(edge_index[0], edge_index[1])

    bw1_p = _pad_lanes(bw1)
    bb1_p = _pad_lanes(bb1[None, :])
    bw2_p = jnp.pad(bw2, ((0, _W - _BH), (0, _W - 2 * _L)))
    bb2_p = _pad_lanes(bb2[None, :])
    linw_p = _pad_lanes(linw)
    linb_p = _pad_lanes(linb[None, :])

    g1w = jnp.stack([lp['gin_w1'][0] for lp in lyr])
    g1b = jnp.stack([lp['gin_w1'][1] for lp in lyr])[:, None, :]
    g2w = jnp.stack([lp['gin_w2'][0] for lp in lyr])
    g2b = jnp.stack([lp['gin_w2'][1] for lp in lyr])[:, None, :]
    eps = jnp.stack([lp['eps'] for lp in lyr]).reshape(1, _L)
    sw = jnp.stack([lp['score'][0][:, 0] for lp in lyr])[:, None, :]

    wq = jnp.stack([lp['wq'][0] for lp in lyr])
    bq = jnp.stack([lp['wq'][1] for lp in lyr])[:, None, :]
    wk = jnp.stack([lp['wk'][0] for lp in lyr])
    bk = jnp.stack([lp['wk'][1] for lp in lyr])[:, None, :]
    wv = jnp.stack([lp['wv'][0] for lp in lyr])
    bv = jnp.stack([lp['wv'][1] for lp in lyr])[:, None, :]
    wo = jnp.stack([lp['wo'][0] for lp in lyr])
    bo = jnp.stack([lp['wo'][1] for lp in lyr])[:, None, :]

    operands = (
        x_hi, x_lo, pe_hi, pe_lo, counts,
        _split2(embw), embb[None, :], _split2(pew_p), peb[None, :],
        _split2(bw1_p), bb1_p, _split2(bw2_p), bb2_p,
        _split2(g1w), g1b, _split2(g2w), g2b, eps,
        sw,
        _split2(wq), bq, _split2(wk), bk, _split2(wv), bv,
        _split2(wo), bo,
        _split2(linw_p), linb_p,
    )

    in_specs = [
        pl.BlockSpec((_R, _FEA), lambda i: (i, 0)),
        pl.BlockSpec((_R, _FEA), lambda i: (i, 0)),
        pl.BlockSpec((_R, 128), lambda i: (i, 0)),
        pl.BlockSpec((_R, 128), lambda i: (i, 0)),
        pl.BlockSpec((_R, _NPG), lambda i: (i, 0)),
    ] + [_full(op.shape) for op in operands[5:]]

    out_shape = (
        jax.ShapeDtypeStruct((_NB, _GPB, _W), f32),
        jax.ShapeDtypeStruct((_NB, _GPB, _W), f32),
        jax.ShapeDtypeStruct((_NB, _GPB, _W), f32),
    )
    out_specs = (
        pl.BlockSpec((1, _GPB, _W), lambda i: (i, 0, 0)),
        pl.BlockSpec((1, _GPB, _W), lambda i: (i, 0, 0)),
        pl.BlockSpec((1, _GPB, _W), lambda i: (i, 0, 0)),
    )

    logits_p, tr_p, lg_p = pl.pallas_call(
        _fused_body,
        grid=(_NB,),
        in_specs=in_specs,
        out_specs=out_specs,
        out_shape=out_shape,
    )(*operands)

    logits = logits_p.reshape(_G, _W)[:, :_T]
    tr = tr_p.reshape(_G, _W)[:, :_L]
    lg = lg_p.reshape(_G, _W)[:, _L:2 * _L]

    costs = [tr[:, l].mean() ** 2 * lg[:, l].mean() for l in range(_L)]
    dense_macs = float(_G * _H * _NPG * _NPG * _DH * 2 + 6 * _N * _C * _C)
    avg_compute = sum(costs) / float(_L)
    total_actual = sum(costs) * dense_macs
    total_dense = jnp.float32(dense_macs * _L)
    return (logits, avg_compute, tr, lg, total_actual, total_dense)


# speedup vs baseline: 2.0642x; 1.5376x over previous
import functools

import jax
import jax.numpy as jnp
import numpy as np
from jax import lax
from jax.experimental import pallas as pl
from jax.experimental.pallas import tpu as pltpu
from jax.experimental.pallas import tpu_sc as plsc

_G = 64
_NPG = 32
_N = _G * _NPG
_EPG = 512
_E = _G * _EPG
_FEA = 128
_C = 256
_L = 4
_T = 10
_H = 8
_DH = _C // _H
_PED = 20
_BH = 64
_MINR = 0.1

_NB = 8
_GPB = _G // _NB
_R = _GPB * _NPG
_W = 128

_dot = functools.partial(lax.dot_general, preferred_element_type=jnp.float32,
                         precision=lax.Precision.HIGHEST)
_dot_fast = functools.partial(lax.dot_general, preferred_element_type=jnp.float32)

_NW = 32
_GPW = _G // _NW
_EPW = _GPW * _EPG
_BINS = _GPW * _NPG * _NPG
_HALF = 16 * _BINS
_IR = 8
_IC = _EPW // _IR


def _hist_body(src_hbm, dst_hbm, out_hbm, src_v, dst_v, idx_v, ones_v,
               zero_v, shared):
    c = lax.axis_index("c")
    s = lax.axis_index("s")
    w = c * 16 + s
    pltpu.sync_copy(src_hbm.at[pl.ds(w * _EPW, _EPW)], src_v)
    pltpu.sync_copy(dst_hbm.at[pl.ds(w * _EPW, _EPW)], dst_v)
    zeros16 = jnp.zeros((16,), jnp.float32)
    ones16 = jnp.ones((16,), jnp.float32)

    def zfill(t, carry):
        zero_v[pl.ds(t * 16, 16)] = zeros16
        return carry

    lax.fori_loop(0, _BINS // 16, zfill, 0)

    def ofill(t, carry):
        ones_v[pl.ds(t * 16, 16)] = ones16
        return carry

    lax.fori_loop(0, _IC // 16, ofill, 0)

    base_row = c * (16 * _GPW * _NPG)
    for r in range(_IR):
        def cbody(k, carry, _r=r):
            sv = src_v[pl.ds(_r * _IC + k * 16, 16)]
            dv = dst_v[pl.ds(_r * _IC + k * 16, 16)]
            idx_v[_r, pl.ds(k * 16, 16)] = ((dv - base_row) * _NPG
                                            + (sv & (_NPG - 1)))
            return carry

        lax.fori_loop(0, _IC // 16, cbody, 0)

    pltpu.sync_copy(zero_v, shared.at[pl.ds(s * _BINS, _BINS)])
    plsc.subcore_barrier()
    for r in range(_IR):
        pltpu.sync_copy(ones_v, shared.at[idx_v.at[r]], add=True)
    plsc.subcore_barrier()
    pltpu.sync_copy(shared.at[pl.ds(s * _BINS, _BINS)],
                    out_hbm.at[pl.ds(c * _HALF + s * _BINS, _BINS)])


def _edge_counts(src, dst):
    mesh = plsc.VectorSubcoreMesh(core_axis_name="c", subcore_axis_name="s")
    run = pl.kernel(
        _hist_body,
        mesh=mesh,
        out_type=jax.ShapeDtypeStruct((2 * _HALF,), jnp.float32),
        scratch_types=[
            pltpu.VMEM((_EPW,), jnp.int32),
            pltpu.VMEM((_EPW,), jnp.int32),
            pltpu.VMEM((_IR, _IC), jnp.int32),
            pltpu.VMEM((_IC,), jnp.float32),
            pltpu.VMEM((_BINS,), jnp.float32),
            pltpu.VMEM_SHARED((_HALF,), jnp.float32),
        ],
    )
    return run(src, dst).reshape(_N, _NPG)


_BF = jnp.bfloat16
_CT = (((1,), (0,)), ((), ()))
_CTT = (((1,), (1,)), ((), ()))


def _sp(a):
    hi = a.astype(_BF)
    lo = (a - hi.astype(jnp.float32)).astype(_BF)
    return hi, lo


def _d3(ahi, alo, whi, wlo, dims=_CT):
    return (_dot_fast(ahi, whi, dims) + _dot_fast(ahi, wlo, dims)
            + _dot_fast(alo, whi, dims))


def _fused_body(xh_ref, xl_ref, peh_ref, pel_ref, cnt_ref,
                embw_ref, embb_ref, pew_ref, peb_ref,
                bw1_ref, bb1_ref, bw2_ref, bb2_ref,
                g1w_ref, g1b_ref, g2w_ref, g2b_ref, eps_ref,
                sw_ref,
                wq_ref, bq_ref, wk_ref, bk_ref, wv_ref, bv_ref,
                wo_ref, bo_ref,
                linw_ref, linb_ref,
                logits_ref, tr_ref, lg_ref):
    f32 = jnp.float32

    row_id = lax.broadcasted_iota(jnp.int32, (_R, _R), 0)
    col_id = lax.broadcasted_iota(jnp.int32, (_R, _R), 1)
    same = (row_id // _NPG) == (col_id // _NPG)
    same_f = same.astype(f32)
    same_b = same.astype(_BF)
    ones_rr = jnp.ones((_R, _R), f32)
    p_sum = ((lax.broadcasted_iota(jnp.int32, (_GPB, _R), 1) // _NPG)
             == lax.broadcasted_iota(jnp.int32, (_GPB, _R), 0)).astype(f32)
    lane_w = lax.broadcasted_iota(jnp.int32, (1, _W), 1)

    ones_b = ones_rr.astype(_BF)
    eye_f = (row_id == col_id).astype(f32)

    def _bc_exact(col):
        d = eye_f * col
        hi = d.astype(_BF)
        r1 = d - hi.astype(jnp.float32)
        mid = r1.astype(_BF)
        lo = (r1 - mid.astype(jnp.float32)).astype(_BF)
        return ((_dot_fast(ones_b, hi, _CT) + _dot_fast(ones_b, mid, _CT))
                + _dot_fast(ones_b, lo, _CT))

    def _bc_int(col):
        return _dot_fast(ones_b, (eye_f * col).astype(_BF), _CT)

    def _bc_approx(col):
        d = eye_f * col
        hi = d.astype(_BF)
        lo = (d - hi.astype(jnp.float32)).astype(_BF)
        return _dot_fast(ones_b, hi, _CT) + _dot_fast(ones_b, lo, _CT)

    h = (_d3(xh_ref[...], xl_ref[...], embw_ref[0], embw_ref[1]) + embb_ref[...]
         + _d3(peh_ref[...], pel_ref[...], pew_ref[0], pew_ref[1])
         + peb_ref[...])

    h_hi, h_lo = _sp(h)
    pooled = (_dot_fast(same_b, h_hi, _CT)
              + _dot_fast(same_b, h_lo, _CT)) * (1.0 / _NPG)
    hid = jnp.maximum(
        _d3(*_sp(pooled), bw1_ref[0], bw1_ref[1]) + bb1_ref[...], 0.0)
    bd = _d3(*_sp(hid), bw2_ref[0], bw2_ref[1]) + bb2_ref[...]
    trs = _MINR + (1.0 - _MINR) * jax.nn.sigmoid(bd)
    lgs = jax.nn.sigmoid(bd)
    tr_ref[...] = (_dot(p_sum, trs, _CT) * (1.0 / _NPG))[None]
    lg_ref[...] = (_dot(p_sum, lgs, _CT) * (1.0 / _NPG))[None]

    tmat = ((lax.broadcasted_iota(jnp.int32, (_NPG, _R), 1) % _NPG)
            == lax.broadcasted_iota(jnp.int32, (_NPG, _R), 0)).astype(_BF)
    cnt_hi, cnt_lo = _sp(cnt_ref[...])
    adj = (_dot_fast(cnt_hi, tmat, _CT)
           + _dot_fast(cnt_lo, tmat, _CT)) * same_f
    adj_hi, adj_lo = _sp(adj)

    neg = jnp.float32(-1e30)
    inv_sqrt = jnp.float32(1.0 / np.sqrt(_DH))

    for l in range(_L):
        agg = _d3(adj_hi, adj_lo, h_hi, h_lo)
        eps = eps_ref[0:1, l:l + 1]
        z = h + eps * h + agg
        t1 = jnp.maximum(
            _d3(*_sp(z), g1w_ref[0, l], g1w_ref[1, l]) + g1b_ref[l], 0.0)
        h_local = (h + _d3(*_sp(t1), g2w_ref[0, l], g2w_ref[1, l])
                   + g2b_ref[l])

        s_col = jnp.sum(h * sw_ref[l], axis=1, keepdims=True)
        smat = _bc_exact(s_col)
        lt = smat < s_col
        eq = smat == s_col
        rank_col = jnp.sum(
            (same & (lt | (eq & (col_id < row_id)))).astype(f32),
            axis=1, keepdims=True)

        trl = jnp.sum(trs * (lane_w == l).astype(f32), axis=1, keepdims=True)
        lgl = jnp.sum(lgs * (lane_w == (_L + l)).astype(f32), axis=1, keepdims=True)
        idx_col = jnp.clip(jnp.floor((1.0 - trl) * float(_NPG - 1)),
                           0.0, float(_NPG - 1))
        dmat = _bc_int(rank_col - idx_col)
        selmat = (dmat == 0.0).astype(f32)
        thr_col = jnp.sum(same_f * smat * selmat, axis=1, keepdims=True)
        m_col = jax.nn.sigmoid(s_col - thr_col)

        bias = jnp.where(same, _bc_approx(jnp.log(m_col + 1e-6)), neg)
        q = _d3(h_hi, h_lo, wq_ref[0, l], wq_ref[1, l]) + bq_ref[l]
        k = _d3(h_hi, h_lo, wk_ref[0, l], wk_ref[1, l]) + bk_ref[l]
        v = _d3(h_hi, h_lo, wv_ref[0, l], wv_ref[1, l]) + bv_ref[l]
        q_hi, q_lo = _sp(q)
        k_hi, k_lo = _sp(k)
        v_hi, v_lo = _sp(v)
        ovs = []
        for hh in range(_H):
            sl = slice(hh * _DH, (hh + 1) * _DH)
            sc = (_dot_fast(q_hi[:, sl], k_hi[:, sl], _CTT)
                  + _dot_fast(q_hi[:, sl], k_lo[:, sl], _CTT)
                  + _dot_fast(q_lo[:, sl], k_hi[:, sl], _CTT)) * inv_sqrt + bias
            p = jax.nn.softmax(sc, axis=-1)
            p_hi, p_lo = _sp(p)
            ovs.append(_dot_fast(p_hi, v_hi[:, sl], _CT)
                       + _dot_fast(p_hi, v_lo[:, sl], _CT)
                       + _dot_fast(p_lo, v_hi[:, sl], _CT))
        o_mat = jnp.concatenate(ovs, axis=1)
        o = (_d3(*_sp(o_mat), wo_ref[0, l], wo_ref[1, l])
             + bo_ref[l]) * m_col
        hsum = h_local + h + lgl * o
        mu = jnp.mean(hsum, axis=-1, keepdims=True)
        d = hsum - mu
        var = jnp.mean(d * d, axis=-1, keepdims=True)
        h = d / jnp.sqrt(var + 1e-5)
        h_hi, h_lo = _sp(h)

    out = _d3(h_hi, h_lo, linw_ref[0], linw_ref[1])
    logits_ref[...] = (_dot(p_sum, out, _CT) + linb_ref[...])[None]


def _full(shape):
    nd = len(shape)
    return pl.BlockSpec(shape, lambda i, _nd=nd: (0,) * _nd)


def _pad_lanes(a, w=_W):
    return jnp.pad(a, ((0, 0),) * (a.ndim - 1) + ((0, w - a.shape[-1]),))


def _split2(a):
    hi = a.astype(_BF)
    lo = (a - hi.astype(jnp.float32)).astype(_BF)
    return jnp.stack([hi, lo])


def kernel(x, pe, edge_index, edge_attr, batch, params):
    del edge_attr, batch
    f32 = jnp.float32
    lyr = params['layers']

    embw, embb = params['node_emb']
    pew, peb = params['pe_lin']
    bw1, bb1 = params['budget_w1']
    bw2, bb2 = params['budget_w2']
    linw, linb = params['lin']

    x_hi = x.astype(_BF)
    x_lo = (x - x_hi.astype(f32)).astype(_BF)
    pe_p = _pad_lanes(pe)
    pe_hi = pe_p.astype(_BF)
    pe_lo = (pe_p - pe_hi.astype(f32)).astype(_BF)
    pew_p = jnp.pad(pew, ((0, 128 - _PED), (0, 0)))
    counts = _edge_counts(edge_index[0], edge_index[1])

    bw1_p = _pad_lanes(bw1)
    bb1_p = _pad_lanes(bb1[None, :])
    bw2_p = jnp.pad(bw2, ((0, _W - _BH), (0, _W - 2 * _L)))
    bb2_p = _pad_lanes(bb2[None, :])
    linw_p = _pad_lanes(linw)
    linb_p = _pad_lanes(linb[None, :])

    g1w = jnp.stack([lp['gin_w1'][0] for lp in lyr])
    g1b = jnp.stack([lp['gin_w1'][1] for lp in lyr])[:, None, :]
    g2w = jnp.stack([lp['gin_w2'][0] for lp in lyr])
    g2b = jnp.stack([lp['gin_w2'][1] for lp in lyr])[:, None, :]
    eps = jnp.stack([lp['eps'] for lp in lyr]).reshape(1, _L)
    sw = jnp.stack([lp['score'][0][:, 0] for lp in lyr])[:, None, :]

    wq = jnp.stack([lp['wq'][0] for lp in lyr])
    bq = jnp.stack([lp['wq'][1] for lp in lyr])[:, None, :]
    wk = jnp.stack([lp['wk'][0] for lp in lyr])
    bk = jnp.stack([lp['wk'][1] for lp in lyr])[:, None, :]
    wv = jnp.stack([lp['wv'][0] for lp in lyr])
    bv = jnp.stack([lp['wv'][1] for lp in lyr])[:, None, :]
    wo = jnp.stack([lp['wo'][0] for lp in lyr])
    bo = jnp.stack([lp['wo'][1] for lp in lyr])[:, None, :]

    operands = (
        x_hi, x_lo, pe_hi, pe_lo, counts,
        _split2(embw), embb[None, :], _split2(pew_p), peb[None, :],
        _split2(bw1_p), bb1_p, _split2(bw2_p), bb2_p,
        _split2(g1w), g1b, _split2(g2w), g2b, eps,
        sw,
        _split2(wq), bq, _split2(wk), bk, _split2(wv), bv,
        _split2(wo), bo,
        _split2(linw_p), linb_p,
    )

    in_specs = [
        pl.BlockSpec((_R, _FEA), lambda i: (i, 0)),
        pl.BlockSpec((_R, _FEA), lambda i: (i, 0)),
        pl.BlockSpec((_R, 128), lambda i: (i, 0)),
        pl.BlockSpec((_R, 128), lambda i: (i, 0)),
        pl.BlockSpec((_R, _NPG), lambda i: (i, 0)),
    ] + [_full(op.shape) for op in operands[5:]]

    out_shape = (
        jax.ShapeDtypeStruct((_NB, _GPB, _W), f32),
        jax.ShapeDtypeStruct((_NB, _GPB, _W), f32),
        jax.ShapeDtypeStruct((_NB, _GPB, _W), f32),
    )
    out_specs = (
        pl.BlockSpec((1, _GPB, _W), lambda i: (i, 0, 0)),
        pl.BlockSpec((1, _GPB, _W), lambda i: (i, 0, 0)),
        pl.BlockSpec((1, _GPB, _W), lambda i: (i, 0, 0)),
    )

    logits_p, tr_p, lg_p = pl.pallas_call(
        _fused_body,
        grid=(_NB,),
        in_specs=in_specs,
        out_specs=out_specs,
        out_shape=out_shape,
    )(*operands)

    logits = logits_p.reshape(_G, _W)[:, :_T]
    tr = tr_p.reshape(_G, _W)[:, :_L]
    lg = lg_p.reshape(_G, _W)[:, _L:2 * _L]

    costs = [tr[:, l].mean() ** 2 * lg[:, l].mean() for l in range(_L)]
    dense_macs = float(_G * _H * _NPG * _NPG * _DH * 2 + 6 * _N * _C * _C)
    avg_compute = sum(costs) / float(_L)
    total_actual = sum(costs) * dense_macs
    total_dense = jnp.float32(dense_macs * _L)
    return (logits, avg_compute, tr, lg, total_actual, total_dense)
